# Initial kernel scaffold; baseline (speedup 1.0000x reference)
import jax
import jax.numpy as jnp
from jax import lax
from jax.experimental import pallas as pl
from jax.experimental.pallas import tpu as pltpu
from jax.experimental.pallas import tpu_sc as plsc

_DIM = 32
_N_EMBED = 8192
_BETA = 0.25

_ROWS = 16384
_BLK = 512
_NB = _ROWS // _BLK

_NC = 2
_NS = 16
_NW = _NC * _NS
_B_PER_W = _ROWS // _NW


def _dist_argmin_body(x_ref, e_ref, ind_ref, acc_ref):
    i = pl.program_id(0)
    x = x_ref[...]
    e = e_ref[...]
    mm = lax.dot_general(x, e, (((1,), (0,)), ((), ())),
                         preferred_element_type=jnp.float32)
    x2 = jnp.sum(x ** 2, axis=1, keepdims=True)
    e2 = jnp.sum(e ** 2, axis=0, keepdims=True)
    dist = x2 - 2.0 * mm + e2
    ind_ref[0, 0, :] = jnp.argmin(dist, axis=1).astype(jnp.int32)

    @pl.when(i == 0)
    def _():
        acc_ref[0, 0] = 0.0

    acc_ref[0, 0] += jnp.sum(jnp.min(dist, axis=1))


def _dist_argmin(flatten, embed):
    return pl.pallas_call(
        _dist_argmin_body,
        grid=(_NB,),
        in_specs=[
            pl.BlockSpec((_BLK, _DIM), lambda i: (i, 0)),
            pl.BlockSpec((_DIM, _N_EMBED), lambda i: (0, 0)),
        ],
        out_specs=[
            pl.BlockSpec((1, 1, _BLK), lambda i: (i, 0, 0)),
            pl.BlockSpec((1, 1), lambda i: (0, 0)),
        ],
        out_shape=[
            jax.ShapeDtypeStruct((_NB, 1, _BLK), jnp.int32),
            jax.ShapeDtypeStruct((1, 1), jnp.float32),
        ],
    )(flatten, embed)


def _gather_body(table_hbm, idx_hbm, out_hbm, idx_v, rows_v, sem):
    wid = lax.axis_index("s") * _NC + lax.axis_index("c")
    base = wid * _B_PER_W
    pltpu.sync_copy(idx_hbm.at[pl.ds(base, _B_PER_W)], idx_v)
    pltpu.async_copy(table_hbm.at[idx_v], rows_v, sem).wait()
    pltpu.sync_copy(rows_v, out_hbm.at[pl.ds(base, _B_PER_W)])


def _sc_gather(table, idx):
    mesh = plsc.VectorSubcoreMesh(core_axis_name="c", subcore_axis_name="s")
    return pl.kernel(
        _gather_body,
        mesh=mesh,
        out_type=jax.ShapeDtypeStruct((_ROWS, _DIM), jnp.float32),
        scratch_types=[
            pltpu.VMEM((_B_PER_W,), jnp.int32),
            pltpu.VMEM((_B_PER_W, _DIM), jnp.float32),
            pltpu.SemaphoreType.DMA,
        ],
    )(table, idx)


def kernel(input, embed):
    flatten = input.reshape(-1, _DIM)
    ind3, acc = _dist_argmin(flatten, embed)
    ind = ind3.reshape(_ROWS)
    quantize = _sc_gather(embed.T, ind)
    mse = acc[0, 0] / float(_ROWS * _DIM)
    diff = mse + mse * _BETA
    quantize_out = quantize.reshape(input.shape)
    embed_ind = ind.reshape(input.shape[:-1])
    return quantize_out, diff, embed_ind


# speedup vs baseline: 1.3642x; 1.3642x over previous
import jax
import jax.numpy as jnp
from jax import lax
from jax.experimental import pallas as pl
from jax.experimental.pallas import tpu as pltpu
from jax.experimental.pallas import tpu_sc as plsc

_DIM = 32
_N_EMBED = 8192
_BETA = 0.25

_ROWS = 16384
_BLK = 512
_NB = _ROWS // _BLK

_NC = 2
_NS = 16
_NW = _NC * _NS
_B_PER_W = _ROWS // _NW


def _dist_argmin_body(x_ref, e_ref, ind_ref, acc_ref):
    i = pl.program_id(0)
    x = x_ref[...]
    e = e_ref[...]
    mm = lax.dot_general(x.astype(jnp.bfloat16), e.astype(jnp.bfloat16),
                         (((1,), (0,)), ((), ())),
                         preferred_element_type=jnp.float32)
    x2 = jnp.sum(x ** 2, axis=1, keepdims=True)
    e2 = jnp.sum(e ** 2, axis=0, keepdims=True)
    dist = x2 - 2.0 * mm + e2
    minv = jnp.min(dist, axis=1, keepdims=True)
    iota = lax.broadcasted_iota(jnp.int32, dist.shape, 1)
    ind = jnp.min(jnp.where(dist == minv, iota, jnp.int32(2**31 - 1)), axis=1)
    ind_ref[0, 0, :] = ind

    @pl.when(i == 0)
    def _():
        acc_ref[0, 0] = 0.0

    acc_ref[0, 0] += jnp.sum(minv)


def _dist_argmin(flatten, embed):
    return pl.pallas_call(
        _dist_argmin_body,
        grid=(_NB,),
        in_specs=[
            pl.BlockSpec((_BLK, _DIM), lambda i: (i, 0)),
            pl.BlockSpec((_DIM, _N_EMBED), lambda i: (0, 0)),
        ],
        out_specs=[
            pl.BlockSpec((1, 1, _BLK), lambda i: (i, 0, 0)),
            pl.BlockSpec(memory_space=pltpu.SMEM),
        ],
        out_shape=[
            jax.ShapeDtypeStruct((_NB, 1, _BLK), jnp.int32),
            jax.ShapeDtypeStruct((1, 1), jnp.float32),
        ],
    )(flatten, embed)


_PAD = 128
_CHUNKS = _B_PER_W // _PAD


def _gather_body(table_hbm, idx_hbm, out_hbm, idx_v, rows_v, sem):
    wid = lax.axis_index("s") * _NC + lax.axis_index("c")
    pltpu.sync_copy(idx_hbm.at[pl.ds(wid * _CHUNKS, _CHUNKS)], idx_v)
    for k in range(_CHUNKS):
        pltpu.async_copy(table_hbm.at[idx_v.at[k]],
                         rows_v.at[pl.ds(k * _PAD, _PAD)], sem)
    pltpu.make_async_copy(table_hbm.at[pl.ds(0, _B_PER_W)], rows_v, sem).wait()
    pltpu.sync_copy(rows_v, out_hbm.at[pl.ds(wid * _B_PER_W, _B_PER_W)])


def _sc_gather(table_pad, idx2d):
    mesh = plsc.VectorSubcoreMesh(core_axis_name="c", subcore_axis_name="s")
    return pl.kernel(
        _gather_body,
        mesh=mesh,
        out_type=jax.ShapeDtypeStruct((_ROWS, _PAD), jnp.float32),
        scratch_types=[
            pltpu.VMEM((_CHUNKS, _PAD), jnp.int32),
            pltpu.VMEM((_B_PER_W, _PAD), jnp.float32),
            pltpu.SemaphoreType.DMA,
        ],
    )(table_pad, idx2d)


def kernel(input, embed):
    flatten = input.reshape(-1, _DIM)
    ind3, acc = _dist_argmin(flatten, embed)
    ind = ind3.reshape(_ROWS)
    table_pad = jnp.pad(embed.T, ((0, 0), (0, _PAD - _DIM)))
    quantize = _sc_gather(table_pad, ind.reshape(-1, _PAD))[:, :_DIM]
    mse = acc[0, 0] / float(_ROWS * _DIM)
    diff = mse + mse * _BETA
    quantize_out = quantize.reshape(input.shape)
    embed_ind = ind.reshape(input.shape[:-1])
    return quantize_out, diff, embed_ind


# speedup vs baseline: 1.3934x; 1.0214x over previous
import jax
import jax.numpy as jnp
from jax import lax
from jax.experimental import pallas as pl
from jax.experimental.pallas import tpu as pltpu
from jax.experimental.pallas import tpu_sc as plsc

_DIM = 32
_N_EMBED = 8192
_BETA = 0.25

_ROWS = 16384
_BLK = 512
_NB = _ROWS // _BLK

_NC = 2
_NS = 16
_NW = _NC * _NS
_B_PER_W = _ROWS // _NW


def _dist_argmin_body(x_ref, e_ref, ind_ref, acc_ref):
    i = pl.program_id(0)
    x = x_ref[...]
    e = e_ref[...]
    mm = lax.dot_general(x.astype(jnp.bfloat16), e.astype(jnp.bfloat16),
                         (((1,), (0,)), ((), ())),
                         preferred_element_type=jnp.float32)
    x2 = jnp.sum(x ** 2, axis=1, keepdims=True)
    e2 = jnp.sum(e ** 2, axis=0, keepdims=True)
    dist = x2 - 2.0 * mm + e2
    minv = jnp.min(dist, axis=1, keepdims=True)
    ind_ref[0, 0, :] = jnp.argmin(dist, axis=1).astype(jnp.int32)

    @pl.when(i == 0)
    def _():
        acc_ref[0, 0] = 0.0

    acc_ref[0, 0] += jnp.sum(minv)


def _dist_argmin(flatten, embed):
    return pl.pallas_call(
        _dist_argmin_body,
        grid=(_NB,),
        in_specs=[
            pl.BlockSpec((_BLK, _DIM), lambda i: (i, 0)),
            pl.BlockSpec((_DIM, _N_EMBED), lambda i: (0, 0)),
        ],
        out_specs=[
            pl.BlockSpec((1, 1, _BLK), lambda i: (i, 0, 0)),
            pl.BlockSpec(memory_space=pltpu.SMEM),
        ],
        out_shape=[
            jax.ShapeDtypeStruct((_NB, 1, _BLK), jnp.int32),
            jax.ShapeDtypeStruct((1, 1), jnp.float32),
        ],
    )(flatten, embed)


_PAD = 128
_CHUNKS = _B_PER_W // _PAD


def _gather_body(table_hbm, idx_hbm, out_hbm, idx_v, rows_v, sem):
    wid = lax.axis_index("s") * _NC + lax.axis_index("c")
    pltpu.sync_copy(idx_hbm.at[pl.ds(wid * _CHUNKS, _CHUNKS)], idx_v)
    for k in range(_CHUNKS):
        pltpu.async_copy(table_hbm.at[idx_v.at[k]],
                         rows_v.at[pl.ds(k * _PAD, _PAD)], sem)
    pltpu.make_async_copy(table_hbm.at[pl.ds(0, _B_PER_W)], rows_v, sem).wait()
    pltpu.sync_copy(rows_v, out_hbm.at[pl.ds(wid * _B_PER_W, _B_PER_W)])


def _sc_gather(table_pad, idx2d):
    mesh = plsc.VectorSubcoreMesh(core_axis_name="c", subcore_axis_name="s")
    return pl.kernel(
        _gather_body,
        mesh=mesh,
        out_type=jax.ShapeDtypeStruct((_ROWS, _PAD), jnp.float32),
        scratch_types=[
            pltpu.VMEM((_CHUNKS, _PAD), jnp.int32),
            pltpu.VMEM((_B_PER_W, _PAD), jnp.float32),
            pltpu.SemaphoreType.DMA,
        ],
    )(table_pad, idx2d)


def kernel(input, embed):
    flatten = input.reshape(-1, _DIM)
    ind3, acc = _dist_argmin(flatten, embed)
    ind = ind3.reshape(_ROWS)
    table_pad = jnp.pad(embed.T, ((0, 0), (0, _PAD - _DIM)))
    quantize = _sc_gather(table_pad, ind.reshape(-1, _PAD))[:, :_DIM]
    mse = acc[0, 0] / float(_ROWS * _DIM)
    diff = mse + mse * _BETA
    quantize_out = quantize.reshape(input.shape)
    embed_ind = ind.reshape(input.shape[:-1])
    return quantize_out, diff, embed_ind
